# baseline (speedup 1.0000x reference)
import functools

import jax
import jax.numpy as jnp
from jax import lax
from jax.experimental import pallas as pl
from jax.experimental.pallas import tpu as pltpu
from jax.experimental.pallas import tpu_sc as plsc

N, C, D = 10000, 32, 128
NW = 32
CH = 4
NCHUNK = N // CH
CPW = -(-NCHUNK // NW)
CPW += CPW % 2
NPAIR = CPW // 2
LAST_BASE = NCHUNK - CPW


def _start_in(in_hbm, buf, sem, chunk):
    pltpu.async_copy(in_hbm.at[pl.ds(chunk * CH, CH)], buf, sem)


def _wait_in(in_hbm, buf, sem):
    pltpu.make_async_copy(in_hbm.at[pl.ds(0, CH)], buf, sem).wait()


def _start_out(out_hbm, acc, sem, chunk):
    pltpu.async_copy(acc, out_hbm.at[pl.ds(chunk * CH, CH)], sem)


def _wait_out(out_hbm, acc, sem):
    pltpu.make_async_copy(acc, out_hbm.at[pl.ds(0, CH)], sem).wait()


def _reduce_chunk(buf, acc):
    for r in range(CH):
        for dc in range(D // 16):
            sl = pl.ds(dc * 16, 16)
            v = buf[r, 0, sl]
            for c in range(1, C):
                v = v + buf[r, c, sl]
            acc[r, sl] = v


def _sc_body(in_hbm, out_hbm, buf0, buf1, acc0, acc1,
             sin0, sin1, sout0, sout1):
    wid = lax.axis_index("s") * 2 + lax.axis_index("c")
    base = jnp.minimum(wid * CPW, LAST_BASE)

    _start_in(in_hbm, buf0, sin0, base)
    _start_in(in_hbm, buf1, sin1, base + 1)

    def pair(j, carry):
        a = base + 2 * j
        b = a + 1

        _wait_in(in_hbm, buf0, sin0)

        @pl.when(j > 0)
        def _():
            _wait_out(out_hbm, acc0, sout0)

        _reduce_chunk(buf0, acc0)

        @pl.when(j < NPAIR - 1)
        def _():
            _start_in(in_hbm, buf0, sin0, a + 2)

        _start_out(out_hbm, acc0, sout0, a)

        _wait_in(in_hbm, buf1, sin1)

        @pl.when(j > 0)
        def _():
            _wait_out(out_hbm, acc1, sout1)

        _reduce_chunk(buf1, acc1)

        @pl.when(j < NPAIR - 1)
        def _():
            _start_in(in_hbm, buf1, sin1, b + 2)

        _start_out(out_hbm, acc1, sout1, b)
        return carry

    lax.fori_loop(0, NPAIR, pair, 0)

    _wait_out(out_hbm, acc0, sout0)
    _wait_out(out_hbm, acc1, sout1)


@jax.jit
def _sum_children(neighbour_states):
    mesh = plsc.VectorSubcoreMesh(core_axis_name="c", subcore_axis_name="s")
    kern = functools.partial(
        pl.kernel,
        out_type=jax.ShapeDtypeStruct((N, D), jnp.float32),
        mesh=mesh,
        scratch_types=[
            pltpu.VMEM((CH, C, D), jnp.float32),
            pltpu.VMEM((CH, C, D), jnp.float32),
            pltpu.VMEM((CH, D), jnp.float32),
            pltpu.VMEM((CH, D), jnp.float32),
            pltpu.SemaphoreType.DMA,
            pltpu.SemaphoreType.DMA,
            pltpu.SemaphoreType.DMA,
            pltpu.SemaphoreType.DMA,
        ],
    )(_sc_body)
    return kern(neighbour_states)


def kernel(neighbour_states):
    return _sum_children(neighbour_states)
